# baseline (speedup 1.0000x reference)
import jax
import jax.numpy as jnp
from jax import lax
from jax.experimental import pallas as pl
from jax.experimental.pallas import tpu as pltpu
from jax.experimental.pallas import tpu_sc as plsc

F32 = jnp.float32
I32 = jnp.int32

N = 50000
NODE_DIM = 14
HID = 64
HEADS = 2
CH = 32

NPAD = 50176
NPT = NPAD // 16
E = 800000
ETOT = E + N
EPT = 53248
EPAD = EPT * 16
CHUNK = 128
NCHUNK = EPT // CHUNK

R = 1024
GRID = NPAD // R


SUP = 128
NSUPER = EPT // SUP


def _sc_edge_body(tab_l, tab_r, src_h, dst_h, att_h,
                  acc_out, s_out,
                  acc_sh, s_sh,
                  xl0, xl1, xr0, xr1, co0,
                  sr0, sr1, dr0, dr1,
                  il0, il1, ir0, ir1,
                  si0, si1, pb0, pb1, att_sm, attd_v,
                  sem_i0, sem_i1, sem_g0, sem_g1, sem_s0, sem_s1):
    c = lax.axis_index("c")
    s = lax.axis_index("s")
    z16f = jnp.zeros((16,), F32)
    iota16 = lax.iota(I32, 16)
    xl_ = (xl0, xl1)
    xr_ = (xr0, xr1)
    co_ = (co0, co0)
    sr_ = (sr0, sr1)
    dr_ = (dr0, dr1)
    il_ = (il0, il1)
    ir_ = (ir0, ir1)
    si_ = (si0, si1)
    pb_ = (pb0, pb1)
    semi = (sem_i0, sem_i1)
    semg = (sem_g0, sem_g1)
    sems = (sem_s0, sem_s1)

    def _z2(i, _):
        co0[i, pl.ds(0, 16)] = z16f
        co0[i, pl.ds(16, 16)] = z16f
        return 0
    lax.fori_loop(0, SUP, _z2, 0)
    for k in range(SUP // 16):
        pb0[pl.ds(k * 16, 16)] = z16f

    row0 = s * NPT

    def _zc(j, _):
        pltpu.sync_copy(co0.at[pl.ds(0, 128)],
                        acc_sh.at[pl.ds(row0 + j * 128, 128)])
        return 0
    lax.fori_loop(0, NPT // 128, _zc, 0)
    pltpu.sync_copy(co0.at[pl.ds(0, NPT % 128)],
                    acc_sh.at[pl.ds(row0 + (NPT // 128) * 128, NPT % 128)])

    def _zs(j, _):
        pltpu.sync_copy(pb0, s_sh.at[pl.ds(row0 + j * 128, 128)])
        return 0
    lax.fori_loop(0, NPT // 128, _zs, 0)
    pltpu.sync_copy(pb0.at[pl.ds(0, NPT % 128)],
                    s_sh.at[pl.ds(row0 + (NPT // 128) * 128, NPT % 128)])

    pltpu.sync_copy(att_h, att_sm)
    plsc.subcore_barrier()
    c16 = jnp.zeros((16,), I32) + c
    for cc in range(CH):
        attd_v[cc, pl.ds(0, 16)] = plsc.load_gather(
            att_sm, [c16, (iota16 + cc) & (CH - 1)])
    ccds = [(iota16 + cc) & (CH - 1) for cc in range(CH)]

    coff = c * NPAD
    rbase0 = s * NSUPER

    def fire_idx(sup, q):
        rb = rbase0 + sup
        pltpu.make_async_copy(src_h.at[pl.ds(rb, 1)], sr_[q], semi[q]).start()
        pltpu.make_async_copy(dst_h.at[pl.ds(rb, 1)], dr_[q], semi[q]).start()

    def wait_idx(q):
        pltpu.make_async_copy(src_h.at[pl.ds(0, 1)], sr_[q], semi[q]).wait()
        pltpu.make_async_copy(dst_h.at[pl.ds(0, 1)], dr_[q], semi[q]).wait()

    def adj(q):
        for k in range(SUP // 16):
            sl = pl.ds(k * 16, 16)
            il_[q][0, sl] = sr_[q][0, sl] + coff
            ir_[q][0, sl] = dr_[q][0, sl] + coff
            si_[q][0, sl] = dr_[q][0, sl]

    def fire_gathers(q):
        pltpu.make_async_copy(tab_l.at[il_[q].at[0]], xl_[q],
                              semg[q]).start()
        pltpu.make_async_copy(tab_r.at[ir_[q].at[0]], xr_[q],
                              semg[q]).start()

    def wait_gathers(q):
        pltpu.make_async_copy(tab_l.at[il_[q].at[0]], xl_[q],
                              semg[q]).wait()
        pltpu.make_async_copy(tab_r.at[ir_[q].at[0]], xr_[q],
                              semg[q]).wait()

    def compute(q):
        xlr = xl_[q]
        xrr = xr_[q]
        cor = co_[q]
        pbr = pb_[q]

        def _grp(g, _):
            row16 = iota16 + g * 16
            acc = [z16f, z16f, z16f, z16f]
            for cc in range(CH):
                ccd = ccds[cc]
                zl = plsc.load_gather(xlr, [row16, ccd])
                zr = plsc.load_gather(xrr, [row16, ccd])
                zz = zl + zr
                lr = jnp.maximum(zz, zz * 0.2)
                acc[cc % 4] = acc[cc % 4] + attd_v[cc, pl.ds(0, 16)] * lr
            p16 = jnp.exp((acc[0] + acc[1]) + (acc[2] + acc[3]))
            pbr[pl.ds(g * 16, 16)] = p16
            for cc in range(CH):
                ccd = ccds[cc]
                gv = plsc.load_gather(xlr, [row16, ccd])
                plsc.store_scatter(cor, [row16, ccd], gv * p16)
            return 0
        lax.fori_loop(0, SUP // 16, _grp, 0, unroll=2)

    def fire_scatters(q):
        pltpu.make_async_copy(co_[q], acc_sh.at[si_[q].at[0]],
                              sems[q]).start(add=True)
        pltpu.make_async_copy(pb_[q], s_sh.at[si_[q].at[0]],
                              sems[q]).start(add=True)

    def drain_scatters(q):
        pltpu.make_async_copy(co_[q], acc_sh.at[si_[q].at[0]],
                              sems[q]).wait()
        pltpu.make_async_copy(pb_[q], s_sh.at[si_[q].at[0]],
                              sems[q]).wait()

    def proc(sup, q):
        @pl.when(sup + 1 < NSUPER)
        def _():
            wait_idx(1 - q)
            adj(1 - q)
            fire_gathers(1 - q)
        wait_gathers(q)

        @pl.when(sup >= 1)
        def _():
            drain_scatters(1 - q)
        compute(q)
        fire_scatters(q)

        @pl.when(sup + 2 < NSUPER)
        def _():
            fire_idx(sup + 2, q)

    fire_idx(0, 0)
    fire_idx(1, 1)
    wait_idx(0)
    adj(0)
    fire_gathers(0)

    def _pair(i, _):
        proc(2 * i, 0)
        proc(2 * i + 1, 1)
        return 0
    lax.fori_loop(0, NSUPER // 2, _pair, 0)
    drain_scatters(1)

    plsc.subcore_barrier()
    pltpu.sync_copy(acc_sh.at[pl.ds(row0, NPT)],
                    acc_out.at[c, pl.ds(row0, NPT)])
    pltpu.sync_copy(s_sh.at[pl.ds(row0, NPT)],
                    s_out.at[c, pl.ds(row0, NPT)])


def _sc_edge(tab_l, tab_r, srcp, dstp, att):
    mesh = plsc.VectorSubcoreMesh(core_axis_name="c", subcore_axis_name="s",
                                  num_cores=2, num_subcores=16)
    return pl.kernel(
        _sc_edge_body,
        out_type=(jax.ShapeDtypeStruct((2, NPAD, CH), F32),
                  jax.ShapeDtypeStruct((2, NPAD), F32)),
        mesh=mesh,
        compiler_params=pltpu.CompilerParams(needs_layout_passes=False,
                                             use_tc_tiling_on_sc=False),
        scratch_types=[
            pltpu.VMEM_SHARED((NPAD, CH), F32),
            pltpu.VMEM_SHARED((NPAD,), F32),
            pltpu.VMEM((SUP, CH), F32),
            pltpu.VMEM((SUP, CH), F32),
            pltpu.VMEM((SUP, CH), F32),
            pltpu.VMEM((SUP, CH), F32),
            pltpu.VMEM((SUP, CH), F32),
            pltpu.VMEM((1, SUP), I32),
            pltpu.VMEM((1, SUP), I32),
            pltpu.VMEM((1, SUP), I32),
            pltpu.VMEM((1, SUP), I32),
            pltpu.VMEM((1, SUP), I32),
            pltpu.VMEM((1, SUP), I32),
            pltpu.VMEM((1, SUP), I32),
            pltpu.VMEM((1, SUP), I32),
            pltpu.VMEM((1, SUP), I32),
            pltpu.VMEM((1, SUP), I32),
            pltpu.VMEM((SUP,), F32),
            pltpu.VMEM((SUP,), F32),
            pltpu.VMEM((2, CH), F32),
            pltpu.VMEM((CH, 16), F32),
            pltpu.SemaphoreType.DMA,
            pltpu.SemaphoreType.DMA,
            pltpu.SemaphoreType.DMA,
            pltpu.SemaphoreType.DMA,
            pltpu.SemaphoreType.DMA,
            pltpu.SemaphoreType.DMA,
        ],
    )(tab_l, tab_r, srcp.reshape(EPAD // SUP, SUP),
      dstp.reshape(EPAD // SUP, SUP), att)


def _ln(x, g, b):
    mu = jnp.mean(x, axis=-1, keepdims=True)
    xc = x - mu
    v = jnp.mean(xc * xc, axis=-1, keepdims=True)
    return xc * jax.lax.rsqrt(v + 1e-5) * g + b


def _tc_a_body(nf, wp, bp, wl, bl, wr, br, xl_o, xr_o):
    y = jnp.dot(nf[...], wp[...], preferred_element_type=F32) + bp[...]
    xl = jnp.dot(y, wl[...], preferred_element_type=F32) + bl[...]
    xr = jnp.dot(y, wr[...], preferred_element_type=F32) + br[...]
    xl_o[0] = xl[:, :CH]
    xl_o[1] = xl[:, CH:]
    xr_o[0] = xr[:, :CH]
    xr_o[1] = xr[:, CH:]


def _tc_a(nf_p, wp, bp, wl, bl, wr, br):
    full = lambda shp: pl.BlockSpec(shp, lambda i: tuple(0 for _ in shp))
    return pl.pallas_call(
        _tc_a_body,
        grid=(GRID,),
        in_specs=[pl.BlockSpec((R, 16), lambda i: (i, 0)),
                  full((16, HID)), full((1, HID)),
                  full((HID, HID)), full((1, HID)),
                  full((HID, HID)), full((1, HID))],
        out_specs=[pl.BlockSpec((2, R, CH), lambda i: (0, i, 0)),
                   pl.BlockSpec((2, R, CH), lambda i: (0, i, 0))],
        out_shape=[jax.ShapeDtypeStruct((2, NPAD, CH), F32),
                   jax.ShapeDtypeStruct((2, NPAD, CH), F32)],
    )(nf_p, wp, bp, wl, bl, wr, br)


def _combine(acc, sden, bias):
    a0 = acc[0]
    a1 = acc[1]
    s0 = sden[0][:, None] + 1e-16
    s1 = sden[1][:, None] + 1e-16
    gat = jnp.concatenate([a0 / s0, a1 / s1], axis=-1) + bias
    return jnp.where(gat > 0, gat, jnp.exp(gat) - 1.0)


def _tc_b_body(acc, sden, bias, g, beta, wl, bl, wr, br, xl_o, xr_o):
    x = _ln(_combine(acc[...], sden[...], bias[...]), g[...], beta[...])
    xl = jnp.dot(x, wl[...], preferred_element_type=F32) + bl[...]
    xr = jnp.dot(x, wr[...], preferred_element_type=F32) + br[...]
    xl_o[0] = xl[:, :CH]
    xl_o[1] = xl[:, CH:]
    xr_o[0] = xr[:, :CH]
    xr_o[1] = xr[:, CH:]


def _tc_b(acc, sden, bias, g, beta, wl, bl, wr, br):
    full = lambda shp: pl.BlockSpec(shp, lambda i: tuple(0 for _ in shp))
    return pl.pallas_call(
        _tc_b_body,
        grid=(GRID,),
        in_specs=[pl.BlockSpec((2, R, CH), lambda i: (0, i, 0)),
                  pl.BlockSpec((2, R), lambda i: (0, i)),
                  full((1, HID)), full((1, HID)), full((1, HID)),
                  full((HID, HID)), full((1, HID)),
                  full((HID, HID)), full((1, HID))],
        out_specs=[pl.BlockSpec((2, R, CH), lambda i: (0, i, 0)),
                   pl.BlockSpec((2, R, CH), lambda i: (0, i, 0))],
        out_shape=[jax.ShapeDtypeStruct((2, NPAD, CH), F32),
                   jax.ShapeDtypeStruct((2, NPAD, CH), F32)],
    )(acc, sden, bias, g, beta, wl, bl, wr, br)


def _tc_c_body(acc, sden, bias, g, beta, wr_i, wz_i, wn_i,
               br_i, bz_i, bn_i, br_h, bz_h, bn_h, wh, bh, out):
    x = _ln(_combine(acc[...], sden[...], bias[...]), g[...], beta[...])
    ir = jnp.dot(x, wr_i[...], preferred_element_type=F32) + br_i[...]
    iz = jnp.dot(x, wz_i[...], preferred_element_type=F32) + bz_i[...]
    inn = jnp.dot(x, wn_i[...], preferred_element_type=F32) + bn_i[...]
    r = jax.nn.sigmoid(ir + br_h[...])
    z = jax.nn.sigmoid(iz + bz_h[...])
    nn_ = jnp.tanh(inn + r * bn_h[...])
    h = (1.0 - z) * nn_
    y = jnp.dot(h, wh[...], preferred_element_type=F32) + bh[...]
    is_relu = lax.broadcasted_iota(I32, y.shape, 1) == 2
    out[...] = jnp.where(is_relu, jnp.maximum(y, 0.0), jax.nn.sigmoid(y))


def _tc_c(acc, sden, bias, g, beta, wr_i, wz_i, wn_i,
          br_i, bz_i, bn_i, br_h, bz_h, bn_h, wh, bh):
    full = lambda shp: pl.BlockSpec(shp, lambda i: tuple(0 for _ in shp))
    return pl.pallas_call(
        _tc_c_body,
        grid=(GRID,),
        in_specs=[pl.BlockSpec((2, R, CH), lambda i: (0, i, 0)),
                  pl.BlockSpec((2, R), lambda i: (0, i)),
                  full((1, HID)), full((1, HID)), full((1, HID)),
                  full((HID, HID)), full((HID, HID)), full((HID, HID)),
                  full((1, HID)), full((1, HID)), full((1, HID)),
                  full((1, HID)), full((1, HID)), full((1, HID)),
                  full((HID, 4)), full((1, 4))],
        out_specs=pl.BlockSpec((R, 4), lambda i: (i, 0)),
        out_shape=jax.ShapeDtypeStruct((NPAD, 4), F32),
    )(acc, sden, bias, g, beta, wr_i, wz_i, wn_i,
      br_i, bz_i, bn_i, br_h, bz_h, bn_h, wh, bh)


def kernel(node_features, edge_index, W_proj, b_proj, Wl1, bl1, Wr1, br1,
           att1, bias1, Wl2, bl2, Wr2, br2, att2, bias2, g1, beta1, g2,
           beta2, W_ih, b_ih, W_hh, b_hh, Wu, bu, Wf, bf, Wo, bo, Wc, bc):
    nf_p = jnp.zeros((NPAD, 16), F32).at[:N, :NODE_DIM].set(node_features)
    wp = jnp.zeros((16, HID), F32).at[:NODE_DIM].set(W_proj.T)

    loops = jnp.arange(N, dtype=I32)
    padi = jnp.full((EPAD - ETOT,), NPAD - 1, I32)
    srcp = jnp.concatenate([edge_index[0], loops, padi])
    dstp = jnp.concatenate([edge_index[1], loops, padi])

    row = lambda v: v.reshape(1, -1)

    xl1, xr1 = _tc_a(nf_p, wp, row(b_proj), Wl1.T, row(bl1), Wr1.T, row(br1))
    acc1, s1 = _sc_edge(xl1.reshape(2 * NPAD, CH), xr1.reshape(2 * NPAD, CH),
                        srcp, dstp, att1)
    xl2, xr2 = _tc_b(acc1, s1, row(bias1), row(g1), row(beta1),
                     Wl2.T, row(bl2), Wr2.T, row(br2))
    acc2, s2 = _sc_edge(xl2.reshape(2 * NPAD, CH), xr2.reshape(2 * NPAD, CH),
                        srcp, dstp, att2)

    wiT = W_ih.T
    bi = b_ih
    bh_ = b_hh
    wh = jnp.concatenate([Wu, Wf, Wo, Wc], axis=0).T
    bhead = jnp.concatenate([bu, bf, bo, bc]).reshape(1, 4)
    out4 = _tc_c(acc2, s2, row(bias2), row(g2), row(beta2),
                 wiT[:, :HID], wiT[:, HID:2 * HID], wiT[:, 2 * HID:],
                 row(bi[:HID]), row(bi[HID:2 * HID]), row(bi[2 * HID:]),
                 row(bh_[:HID]), row(bh_[HID:2 * HID]), row(bh_[2 * HID:]),
                 wh, bhead)

    util = out4[:N, 0:1]
    feas = out4[:N, 1:2]
    ot = out4[:N, 2:3]
    conf = out4[:N, 3:4]
    return (util, feas, ot, conf)


# speedup vs baseline: 1.0011x; 1.0011x over previous
import jax
import jax.numpy as jnp
from jax import lax
from jax.experimental import pallas as pl
from jax.experimental.pallas import tpu as pltpu
from jax.experimental.pallas import tpu_sc as plsc

F32 = jnp.float32
I32 = jnp.int32

N = 50000
NODE_DIM = 14
HID = 64
HEADS = 2
CH = 32

NPAD = 50176
NPT = NPAD // 16
E = 800000
ETOT = E + N
EPT = 53248
EPAD = EPT * 16
CHUNK = 128
NCHUNK = EPT // CHUNK

R = 1024
GRID = NPAD // R


SUP = 128
NSUPER = EPT // SUP


def _sc_edge_body(tab_l, tab_r, sd_h, att_h,
                  acc_out, s_out,
                  acc_sh, s_sh,
                  xl0, xl1, xr0, xr1, co0,
                  sd0, sd1,
                  il0, il1, ir0, ir1,
                  si0, si1, pb0, pb1, att_sm, attd_v,
                  sem_i0, sem_i1, sem_g0, sem_g1, sem_s0, sem_s1):
    c = lax.axis_index("c")
    s = lax.axis_index("s")
    z16f = jnp.zeros((16,), F32)
    iota16 = lax.iota(I32, 16)
    xl_ = (xl0, xl1)
    xr_ = (xr0, xr1)
    co_ = (co0, co0)
    sd_ = (sd0, sd1)
    il_ = (il0, il1)
    ir_ = (ir0, ir1)
    si_ = (si0, si1)
    pb_ = (pb0, pb1)
    semi = (sem_i0, sem_i1)
    semg = (sem_g0, sem_g1)
    sems = (sem_s0, sem_s1)

    def _z2(i, _):
        co0[i, pl.ds(0, 16)] = z16f
        co0[i, pl.ds(16, 16)] = z16f
        return 0
    lax.fori_loop(0, SUP, _z2, 0)
    for k in range(SUP // 16):
        pb0[pl.ds(k * 16, 16)] = z16f

    row0 = s * NPT

    def _zc(j, _):
        pltpu.sync_copy(co0.at[pl.ds(0, 128)],
                        acc_sh.at[pl.ds(row0 + j * 128, 128)])
        return 0
    lax.fori_loop(0, NPT // 128, _zc, 0)
    pltpu.sync_copy(co0.at[pl.ds(0, NPT % 128)],
                    acc_sh.at[pl.ds(row0 + (NPT // 128) * 128, NPT % 128)])

    def _zs(j, _):
        pltpu.sync_copy(pb0, s_sh.at[pl.ds(row0 + j * 128, 128)])
        return 0
    lax.fori_loop(0, NPT // 128, _zs, 0)
    pltpu.sync_copy(pb0.at[pl.ds(0, NPT % 128)],
                    s_sh.at[pl.ds(row0 + (NPT // 128) * 128, NPT % 128)])

    pltpu.sync_copy(att_h, att_sm)
    plsc.subcore_barrier()
    c16 = jnp.zeros((16,), I32) + c
    for cc in range(CH):
        attd_v[cc, pl.ds(0, 16)] = plsc.load_gather(
            att_sm, [c16, (iota16 + cc) & (CH - 1)])

    coff = c * NPAD
    rbase0 = s * NSUPER

    def fire_idx(sup, q):
        rb = rbase0 + sup
        pltpu.make_async_copy(sd_h.at[rb], sd_[q], semi[q]).start()

    def wait_idx(q):
        pltpu.make_async_copy(sd_h.at[0], sd_[q], semi[q]).wait()

    def adj(q):
        for k in range(SUP // 16):
            sl = pl.ds(k * 16, 16)
            il_[q][0, sl] = sd_[q][0, sl] + coff
            ir_[q][0, sl] = sd_[q][1, sl] + coff
            si_[q][0, sl] = sd_[q][1, sl]

    def fire_gathers(q):
        pltpu.make_async_copy(tab_l.at[il_[q].at[0]], xl_[q],
                              semg[q]).start()
        pltpu.make_async_copy(tab_r.at[ir_[q].at[0]], xr_[q],
                              semg[q]).start()

    def wait_gathers(q):
        pltpu.make_async_copy(tab_l.at[il_[q].at[0]], xl_[q],
                              semg[q]).wait()
        pltpu.make_async_copy(tab_r.at[ir_[q].at[0]], xr_[q],
                              semg[q]).wait()

    def compute(q):
        xlr = xl_[q]
        xrr = xr_[q]
        cor = co_[q]
        pbr = pb_[q]

        def _grp(g, _):
            row16 = iota16 + g * 16
            acc = [z16f, z16f, z16f, z16f]
            for cc in range(CH):
                ccd = (iota16 + cc) & (CH - 1)
                zl = plsc.load_gather(xlr, [row16, ccd])
                zr = plsc.load_gather(xrr, [row16, ccd])
                zz = zl + zr
                lr = jnp.maximum(zz, zz * 0.2)
                acc[cc % 4] = acc[cc % 4] + attd_v[cc, pl.ds(0, 16)] * lr
            p16 = jnp.exp((acc[0] + acc[1]) + (acc[2] + acc[3]))
            pbr[pl.ds(g * 16, 16)] = p16
            for cc in range(CH):
                ccd = (iota16 + cc) & (CH - 1)
                gv = plsc.load_gather(xlr, [row16, ccd])
                plsc.store_scatter(cor, [row16, ccd], gv * p16)
            return 0
        lax.fori_loop(0, SUP // 16, _grp, 0)

    def fire_scatters(q):
        pltpu.make_async_copy(co_[q], acc_sh.at[si_[q].at[0]],
                              sems[q]).start(add=True)
        pltpu.make_async_copy(pb_[q], s_sh.at[si_[q].at[0]],
                              sems[q]).start(add=True)

    def drain_scatters(q):
        pltpu.make_async_copy(co_[q], acc_sh.at[si_[q].at[0]],
                              sems[q]).wait()
        pltpu.make_async_copy(pb_[q], s_sh.at[si_[q].at[0]],
                              sems[q]).wait()

    def proc(sup, q):
        @pl.when(sup + 1 < NSUPER)
        def _():
            wait_idx(1 - q)
            adj(1 - q)
            fire_gathers(1 - q)
        wait_gathers(q)

        @pl.when(sup >= 1)
        def _():
            drain_scatters(1 - q)
        compute(q)
        fire_scatters(q)

        @pl.when(sup + 2 < NSUPER)
        def _():
            fire_idx(sup + 2, q)

    fire_idx(0, 0)
    fire_idx(1, 1)
    wait_idx(0)
    adj(0)
    fire_gathers(0)

    def _pair(i, _):
        proc(2 * i, 0)
        proc(2 * i + 1, 1)
        return 0
    lax.fori_loop(0, NSUPER // 2, _pair, 0)
    drain_scatters(1)

    plsc.subcore_barrier()
    pltpu.sync_copy(acc_sh.at[pl.ds(row0, NPT)],
                    acc_out.at[c, pl.ds(row0, NPT)])
    pltpu.sync_copy(s_sh.at[pl.ds(row0, NPT)],
                    s_out.at[c, pl.ds(row0, NPT)])


def _sc_edge(tab_l, tab_r, srcp, dstp, att):
    mesh = plsc.VectorSubcoreMesh(core_axis_name="c", subcore_axis_name="s",
                                  num_cores=2, num_subcores=16)
    return pl.kernel(
        _sc_edge_body,
        out_type=(jax.ShapeDtypeStruct((2, NPAD, CH), F32),
                  jax.ShapeDtypeStruct((2, NPAD), F32)),
        mesh=mesh,
        compiler_params=pltpu.CompilerParams(needs_layout_passes=False,
                                             use_tc_tiling_on_sc=False),
        scratch_types=[
            pltpu.VMEM_SHARED((NPAD, CH), F32),
            pltpu.VMEM_SHARED((NPAD,), F32),
            pltpu.VMEM((SUP, CH), F32),
            pltpu.VMEM((SUP, CH), F32),
            pltpu.VMEM((SUP, CH), F32),
            pltpu.VMEM((SUP, CH), F32),
            pltpu.VMEM((SUP, CH), F32),
            pltpu.VMEM((2, SUP), I32),
            pltpu.VMEM((2, SUP), I32),
            pltpu.VMEM((1, SUP), I32),
            pltpu.VMEM((1, SUP), I32),
            pltpu.VMEM((1, SUP), I32),
            pltpu.VMEM((1, SUP), I32),
            pltpu.VMEM((1, SUP), I32),
            pltpu.VMEM((1, SUP), I32),
            pltpu.VMEM((SUP,), F32),
            pltpu.VMEM((SUP,), F32),
            pltpu.VMEM((2, CH), F32),
            pltpu.VMEM((CH, 16), F32),
            pltpu.SemaphoreType.DMA,
            pltpu.SemaphoreType.DMA,
            pltpu.SemaphoreType.DMA,
            pltpu.SemaphoreType.DMA,
            pltpu.SemaphoreType.DMA,
            pltpu.SemaphoreType.DMA,
        ],
    )(tab_l, tab_r,
      jnp.stack([srcp.reshape(EPAD // SUP, SUP),
                 dstp.reshape(EPAD // SUP, SUP)], axis=1), att)


def _ln(x, g, b):
    mu = jnp.mean(x, axis=-1, keepdims=True)
    xc = x - mu
    v = jnp.mean(xc * xc, axis=-1, keepdims=True)
    return xc * jax.lax.rsqrt(v + 1e-5) * g + b


def _tc_a_body(nf, wp, bp, wl, bl, wr, br, xl_o, xr_o):
    y = jnp.dot(nf[...], wp[...], preferred_element_type=F32) + bp[...]
    xl = jnp.dot(y, wl[...], preferred_element_type=F32) + bl[...]
    xr = jnp.dot(y, wr[...], preferred_element_type=F32) + br[...]
    xl_o[0] = xl[:, :CH]
    xl_o[1] = xl[:, CH:]
    xr_o[0] = xr[:, :CH]
    xr_o[1] = xr[:, CH:]


def _tc_a(nf_p, wp, bp, wl, bl, wr, br):
    full = lambda shp: pl.BlockSpec(shp, lambda i: tuple(0 for _ in shp))
    return pl.pallas_call(
        _tc_a_body,
        grid=(GRID,),
        in_specs=[pl.BlockSpec((R, 16), lambda i: (i, 0)),
                  full((16, HID)), full((1, HID)),
                  full((HID, HID)), full((1, HID)),
                  full((HID, HID)), full((1, HID))],
        out_specs=[pl.BlockSpec((2, R, CH), lambda i: (0, i, 0)),
                   pl.BlockSpec((2, R, CH), lambda i: (0, i, 0))],
        out_shape=[jax.ShapeDtypeStruct((2, NPAD, CH), F32),
                   jax.ShapeDtypeStruct((2, NPAD, CH), F32)],
    )(nf_p, wp, bp, wl, bl, wr, br)


def _combine(acc, sden, bias):
    a0 = acc[0]
    a1 = acc[1]
    s0 = sden[0][:, None] + 1e-16
    s1 = sden[1][:, None] + 1e-16
    gat = jnp.concatenate([a0 / s0, a1 / s1], axis=-1) + bias
    return jnp.where(gat > 0, gat, jnp.exp(gat) - 1.0)


def _tc_b_body(acc, sden, bias, g, beta, wl, bl, wr, br, xl_o, xr_o):
    x = _ln(_combine(acc[...], sden[...], bias[...]), g[...], beta[...])
    xl = jnp.dot(x, wl[...], preferred_element_type=F32) + bl[...]
    xr = jnp.dot(x, wr[...], preferred_element_type=F32) + br[...]
    xl_o[0] = xl[:, :CH]
    xl_o[1] = xl[:, CH:]
    xr_o[0] = xr[:, :CH]
    xr_o[1] = xr[:, CH:]


def _tc_b(acc, sden, bias, g, beta, wl, bl, wr, br):
    full = lambda shp: pl.BlockSpec(shp, lambda i: tuple(0 for _ in shp))
    return pl.pallas_call(
        _tc_b_body,
        grid=(GRID,),
        in_specs=[pl.BlockSpec((2, R, CH), lambda i: (0, i, 0)),
                  pl.BlockSpec((2, R), lambda i: (0, i)),
                  full((1, HID)), full((1, HID)), full((1, HID)),
                  full((HID, HID)), full((1, HID)),
                  full((HID, HID)), full((1, HID))],
        out_specs=[pl.BlockSpec((2, R, CH), lambda i: (0, i, 0)),
                   pl.BlockSpec((2, R, CH), lambda i: (0, i, 0))],
        out_shape=[jax.ShapeDtypeStruct((2, NPAD, CH), F32),
                   jax.ShapeDtypeStruct((2, NPAD, CH), F32)],
    )(acc, sden, bias, g, beta, wl, bl, wr, br)


def _tc_c_body(acc, sden, bias, g, beta, wr_i, wz_i, wn_i,
               br_i, bz_i, bn_i, br_h, bz_h, bn_h, wh, bh, out):
    x = _ln(_combine(acc[...], sden[...], bias[...]), g[...], beta[...])
    ir = jnp.dot(x, wr_i[...], preferred_element_type=F32) + br_i[...]
    iz = jnp.dot(x, wz_i[...], preferred_element_type=F32) + bz_i[...]
    inn = jnp.dot(x, wn_i[...], preferred_element_type=F32) + bn_i[...]
    r = jax.nn.sigmoid(ir + br_h[...])
    z = jax.nn.sigmoid(iz + bz_h[...])
    nn_ = jnp.tanh(inn + r * bn_h[...])
    h = (1.0 - z) * nn_
    y = jnp.dot(h, wh[...], preferred_element_type=F32) + bh[...]
    is_relu = lax.broadcasted_iota(I32, y.shape, 1) == 2
    out[...] = jnp.where(is_relu, jnp.maximum(y, 0.0), jax.nn.sigmoid(y))


def _tc_c(acc, sden, bias, g, beta, wr_i, wz_i, wn_i,
          br_i, bz_i, bn_i, br_h, bz_h, bn_h, wh, bh):
    full = lambda shp: pl.BlockSpec(shp, lambda i: tuple(0 for _ in shp))
    return pl.pallas_call(
        _tc_c_body,
        grid=(GRID,),
        in_specs=[pl.BlockSpec((2, R, CH), lambda i: (0, i, 0)),
                  pl.BlockSpec((2, R), lambda i: (0, i)),
                  full((1, HID)), full((1, HID)), full((1, HID)),
                  full((HID, HID)), full((HID, HID)), full((HID, HID)),
                  full((1, HID)), full((1, HID)), full((1, HID)),
                  full((1, HID)), full((1, HID)), full((1, HID)),
                  full((HID, 4)), full((1, 4))],
        out_specs=pl.BlockSpec((R, 4), lambda i: (i, 0)),
        out_shape=jax.ShapeDtypeStruct((NPAD, 4), F32),
    )(acc, sden, bias, g, beta, wr_i, wz_i, wn_i,
      br_i, bz_i, bn_i, br_h, bz_h, bn_h, wh, bh)


def kernel(node_features, edge_index, W_proj, b_proj, Wl1, bl1, Wr1, br1,
           att1, bias1, Wl2, bl2, Wr2, br2, att2, bias2, g1, beta1, g2,
           beta2, W_ih, b_ih, W_hh, b_hh, Wu, bu, Wf, bf, Wo, bo, Wc, bc):
    nf_p = jnp.zeros((NPAD, 16), F32).at[:N, :NODE_DIM].set(node_features)
    wp = jnp.zeros((16, HID), F32).at[:NODE_DIM].set(W_proj.T)

    loops = jnp.arange(N, dtype=I32)
    padi = jnp.full((EPAD - ETOT,), NPAD - 1, I32)
    srcp = jnp.concatenate([edge_index[0], loops, padi])
    dstp = jnp.concatenate([edge_index[1], loops, padi])

    row = lambda v: v.reshape(1, -1)

    xl1, xr1 = _tc_a(nf_p, wp, row(b_proj), Wl1.T, row(bl1), Wr1.T, row(br1))
    acc1, s1 = _sc_edge(xl1.reshape(2 * NPAD, CH), xr1.reshape(2 * NPAD, CH),
                        srcp, dstp, att1)
    xl2, xr2 = _tc_b(acc1, s1, row(bias1), row(g1), row(beta1),
                     Wl2.T, row(bl2), Wr2.T, row(br2))
    acc2, s2 = _sc_edge(xl2.reshape(2 * NPAD, CH), xr2.reshape(2 * NPAD, CH),
                        srcp, dstp, att2)

    wiT = W_ih.T
    bi = b_ih
    bh_ = b_hh
    wh = jnp.concatenate([Wu, Wf, Wo, Wc], axis=0).T
    bhead = jnp.concatenate([bu, bf, bo, bc]).reshape(1, 4)
    out4 = _tc_c(acc2, s2, row(bias2), row(g2), row(beta2),
                 wiT[:, :HID], wiT[:, HID:2 * HID], wiT[:, 2 * HID:],
                 row(bi[:HID]), row(bi[HID:2 * HID]), row(bi[2 * HID:]),
                 row(bh_[:HID]), row(bh_[HID:2 * HID]), row(bh_[2 * HID:]),
                 wh, bhead)

    util = out4[:N, 0:1]
    feas = out4[:N, 1:2]
    ot = out4[:N, 2:3]
    conf = out4[:N, 3:4]
    return (util, feas, ot, conf)


# speedup vs baseline: 1.0071x; 1.0059x over previous
import jax
import jax.numpy as jnp
from jax import lax
from jax.experimental import pallas as pl
from jax.experimental.pallas import tpu as pltpu
from jax.experimental.pallas import tpu_sc as plsc

F32 = jnp.float32
I32 = jnp.int32

N = 50000
NODE_DIM = 14
HID = 64
HEADS = 2
CH = 32

NPAD = 50176
NPT = NPAD // 16
E = 800000
ETOT = E + N
EPT = 53248
EPAD = EPT * 16
CHUNK = 128
NCHUNK = EPT // CHUNK

R = 1024
GRID = NPAD // R


SUP = 128
NSUPER = EPT // SUP


def _sc_edge_body(tab_l, tab_r, src_h, dst_h, att_h,
                  acc_out, s_out,
                  acc_sh, s_sh,
                  xl0, xl1, xr0, xr1, co0,
                  sr0, sr1, dr0, dr1,
                  il0, il1, ir0, ir1,
                  si0, si1, pb0, pb1, att_sm, attd_v,
                  sem_i0, sem_i1, sem_g0, sem_g1, sem_s0, sem_s1):
    c = lax.axis_index("c")
    s = lax.axis_index("s")
    z16f = jnp.zeros((16,), F32)
    iota16 = lax.iota(I32, 16)
    xl_ = (xl0, xl1)
    xr_ = (xr0, xr1)
    co_ = (co0, co0)
    sr_ = (sr0, sr1)
    dr_ = (dr0, dr1)
    il_ = (il0, il1)
    ir_ = (ir0, ir1)
    si_ = (si0, si1)
    pb_ = (pb0, pb1)
    semi = (sem_i0, sem_i1)
    semg = (sem_g0, sem_g1)
    sems = (sem_s0, sem_s1)

    def _z2(i, _):
        co0[i, pl.ds(0, 16)] = z16f
        co0[i, pl.ds(16, 16)] = z16f
        return 0
    lax.fori_loop(0, SUP, _z2, 0)
    for k in range(SUP // 16):
        pb0[pl.ds(k * 16, 16)] = z16f

    row0 = s * NPT

    def _zc(j, _):
        pltpu.sync_copy(co0.at[pl.ds(0, 128)],
                        acc_sh.at[pl.ds(row0 + j * 128, 128)])
        return 0
    lax.fori_loop(0, NPT // 128, _zc, 0)
    pltpu.sync_copy(co0.at[pl.ds(0, NPT % 128)],
                    acc_sh.at[pl.ds(row0 + (NPT // 128) * 128, NPT % 128)])

    def _zs(j, _):
        pltpu.sync_copy(pb0, s_sh.at[pl.ds(row0 + j * 128, 128)])
        return 0
    lax.fori_loop(0, NPT // 128, _zs, 0)
    pltpu.sync_copy(pb0.at[pl.ds(0, NPT % 128)],
                    s_sh.at[pl.ds(row0 + (NPT // 128) * 128, NPT % 128)])

    pltpu.sync_copy(att_h, att_sm)
    plsc.subcore_barrier()
    c16 = jnp.zeros((16,), I32) + c
    for cc in range(CH):
        attd_v[cc, pl.ds(0, 16)] = plsc.load_gather(
            att_sm, [c16, (iota16 + cc) & (CH - 1)])

    coff = c * NPAD
    rbase0 = s * NSUPER

    def fire_idx(sup, q):
        rb = rbase0 + sup
        pltpu.make_async_copy(src_h.at[pl.ds(rb, 1)], sr_[q], semi[q]).start()
        pltpu.make_async_copy(dst_h.at[pl.ds(rb, 1)], dr_[q], semi[q]).start()

    def wait_idx(q):
        pltpu.make_async_copy(src_h.at[pl.ds(0, 1)], sr_[q], semi[q]).wait()
        pltpu.make_async_copy(dst_h.at[pl.ds(0, 1)], dr_[q], semi[q]).wait()

    def adj(q):
        for k in range(SUP // 16):
            sl = pl.ds(k * 16, 16)
            il_[q][0, sl] = sr_[q][0, sl] + coff
            ir_[q][0, sl] = dr_[q][0, sl] + coff
            si_[q][0, sl] = dr_[q][0, sl]

    def fire_gathers(q):
        pltpu.make_async_copy(tab_l.at[il_[q].at[0]], xl_[q],
                              semg[q]).start()
        pltpu.make_async_copy(tab_r.at[ir_[q].at[0]], xr_[q],
                              semg[q]).start()

    def wait_gathers(q):
        pltpu.make_async_copy(tab_l.at[il_[q].at[0]], xl_[q],
                              semg[q]).wait()
        pltpu.make_async_copy(tab_r.at[ir_[q].at[0]], xr_[q],
                              semg[q]).wait()

    def compute(q):
        xlr = xl_[q]
        xrr = xr_[q]
        cor = co_[q]
        pbr = pb_[q]

        def _grp(g, _):
            row16 = iota16 + g * 16
            acc = [z16f, z16f, z16f, z16f]
            for cc in range(CH):
                ccd = (iota16 + cc) & (CH - 1)
                zl = plsc.load_gather(xlr, [row16, ccd])
                zr = plsc.load_gather(xrr, [row16, ccd])
                zz = zl + zr
                lr = jnp.maximum(zz, zz * 0.2)
                acc[cc % 4] = acc[cc % 4] + attd_v[cc, pl.ds(0, 16)] * lr
            p16 = jnp.exp((acc[0] + acc[1]) + (acc[2] + acc[3]))
            pbr[pl.ds(g * 16, 16)] = p16
            for cc in range(CH):
                ccd = (iota16 + cc) & (CH - 1)
                gv = plsc.load_gather(xlr, [row16, ccd])
                plsc.store_scatter(cor, [row16, ccd], gv * p16)
            return 0
        lax.fori_loop(0, SUP // 16, _grp, 0)

    def fire_scatters(q):
        pltpu.make_async_copy(co_[q], acc_sh.at[si_[q].at[0]],
                              sems[q]).start(add=True)
        pltpu.make_async_copy(pb_[q], s_sh.at[si_[q].at[0]],
                              sems[q]).start(add=True)

    def drain_scatters(q):
        pltpu.make_async_copy(co_[q], acc_sh.at[si_[q].at[0]],
                              sems[q]).wait()
        pltpu.make_async_copy(pb_[q], s_sh.at[si_[q].at[0]],
                              sems[q]).wait()

    def proc(sup, q):
        @pl.when(sup + 1 < NSUPER)
        def _():
            wait_idx(1 - q)
            adj(1 - q)
            fire_gathers(1 - q)
        wait_gathers(q)

        @pl.when(sup >= 1)
        def _():
            drain_scatters(1 - q)
        compute(q)
        fire_scatters(q)

        @pl.when(sup + 2 < NSUPER)
        def _():
            fire_idx(sup + 2, q)

    fire_idx(0, 0)
    fire_idx(1, 1)
    wait_idx(0)
    adj(0)
    fire_gathers(0)

    def _pair(i, _):
        proc(2 * i, 0)
        proc(2 * i + 1, 1)
        return 0
    lax.fori_loop(0, NSUPER // 2, _pair, 0)
    drain_scatters(1)

    plsc.subcore_barrier()
    pltpu.sync_copy(acc_sh.at[pl.ds(row0, NPT)],
                    acc_out.at[c, pl.ds(row0, NPT)])
    pltpu.sync_copy(s_sh.at[pl.ds(row0, NPT)],
                    s_out.at[c, pl.ds(row0, NPT)])


def _sc_edge(tab_l, tab_r, srcp, dstp, att):
    mesh = plsc.VectorSubcoreMesh(core_axis_name="c", subcore_axis_name="s",
                                  num_cores=2, num_subcores=16)
    return pl.kernel(
        _sc_edge_body,
        out_type=(jax.ShapeDtypeStruct((2, NPAD, CH), F32),
                  jax.ShapeDtypeStruct((2, NPAD), F32)),
        mesh=mesh,
        compiler_params=pltpu.CompilerParams(needs_layout_passes=False,
                                             use_tc_tiling_on_sc=False),
        scratch_types=[
            pltpu.VMEM_SHARED((NPAD, CH), F32),
            pltpu.VMEM_SHARED((NPAD,), F32),
            pltpu.VMEM((SUP, CH), F32),
            pltpu.VMEM((SUP, CH), F32),
            pltpu.VMEM((SUP, CH), F32),
            pltpu.VMEM((SUP, CH), F32),
            pltpu.VMEM((SUP, CH), F32),
            pltpu.VMEM((1, SUP), I32),
            pltpu.VMEM((1, SUP), I32),
            pltpu.VMEM((1, SUP), I32),
            pltpu.VMEM((1, SUP), I32),
            pltpu.VMEM((1, SUP), I32),
            pltpu.VMEM((1, SUP), I32),
            pltpu.VMEM((1, SUP), I32),
            pltpu.VMEM((1, SUP), I32),
            pltpu.VMEM((1, SUP), I32),
            pltpu.VMEM((1, SUP), I32),
            pltpu.VMEM((SUP,), F32),
            pltpu.VMEM((SUP,), F32),
            pltpu.VMEM((2, CH), F32),
            pltpu.VMEM((CH, 16), F32),
            pltpu.SemaphoreType.DMA,
            pltpu.SemaphoreType.DMA,
            pltpu.SemaphoreType.DMA,
            pltpu.SemaphoreType.DMA,
            pltpu.SemaphoreType.DMA,
            pltpu.SemaphoreType.DMA,
        ],
    )(tab_l, tab_r, srcp.reshape(EPAD // SUP, SUP),
      dstp.reshape(EPAD // SUP, SUP), att)


def _ln(x, g, b):
    mu = jnp.mean(x, axis=-1, keepdims=True)
    xc = x - mu
    v = jnp.mean(xc * xc, axis=-1, keepdims=True)
    return xc * jax.lax.rsqrt(v + 1e-5) * g + b


def _tc_a_body(nf, wp, bp, wl, bl, wr, br, xl_o, xr_o):
    y = jnp.dot(nf[...], wp[...], preferred_element_type=F32) + bp[...]
    xl = jnp.dot(y, wl[...], preferred_element_type=F32) + bl[...]
    xr = jnp.dot(y, wr[...], preferred_element_type=F32) + br[...]
    xl_o[0] = xl[:, :CH]
    xl_o[1] = xl[:, CH:]
    xr_o[0] = xr[:, :CH]
    xr_o[1] = xr[:, CH:]


def _tc_a(nf_p, wp, bp, wl, bl, wr, br):
    full = lambda shp: pl.BlockSpec(shp, lambda i: tuple(0 for _ in shp))
    return pl.pallas_call(
        _tc_a_body,
        grid=(GRID,),
        in_specs=[pl.BlockSpec((R, 16), lambda i: (i, 0)),
                  full((16, HID)), full((1, HID)),
                  full((HID, HID)), full((1, HID)),
                  full((HID, HID)), full((1, HID))],
        out_specs=[pl.BlockSpec((2, R, CH), lambda i: (0, i, 0)),
                   pl.BlockSpec((2, R, CH), lambda i: (0, i, 0))],
        out_shape=[jax.ShapeDtypeStruct((2, NPAD, CH), F32),
                   jax.ShapeDtypeStruct((2, NPAD, CH), F32)],
    )(nf_p, wp, bp, wl, bl, wr, br)


def _combine(acc, sden, bias):
    a0 = acc[0]
    a1 = acc[1]
    s0 = sden[0][:, None] + 1e-16
    s1 = sden[1][:, None] + 1e-16
    gat = jnp.concatenate([a0 / s0, a1 / s1], axis=-1) + bias
    return jnp.where(gat > 0, gat, jnp.exp(gat) - 1.0)


def _tc_b_body(acc, sden, bias, g, beta, wl, bl, wr, br, xl_o, xr_o):
    x = _ln(_combine(acc[...], sden[...], bias[...]), g[...], beta[...])
    xl = jnp.dot(x, wl[...], preferred_element_type=F32) + bl[...]
    xr = jnp.dot(x, wr[...], preferred_element_type=F32) + br[...]
    xl_o[0] = xl[:, :CH]
    xl_o[1] = xl[:, CH:]
    xr_o[0] = xr[:, :CH]
    xr_o[1] = xr[:, CH:]


def _tc_b(acc, sden, bias, g, beta, wl, bl, wr, br):
    full = lambda shp: pl.BlockSpec(shp, lambda i: tuple(0 for _ in shp))
    return pl.pallas_call(
        _tc_b_body,
        grid=(GRID,),
        in_specs=[pl.BlockSpec((2, R, CH), lambda i: (0, i, 0)),
                  pl.BlockSpec((2, R), lambda i: (0, i)),
                  full((1, HID)), full((1, HID)), full((1, HID)),
                  full((HID, HID)), full((1, HID)),
                  full((HID, HID)), full((1, HID))],
        out_specs=[pl.BlockSpec((2, R, CH), lambda i: (0, i, 0)),
                   pl.BlockSpec((2, R, CH), lambda i: (0, i, 0))],
        out_shape=[jax.ShapeDtypeStruct((2, NPAD, CH), F32),
                   jax.ShapeDtypeStruct((2, NPAD, CH), F32)],
    )(acc, sden, bias, g, beta, wl, bl, wr, br)


def _tc_c_body(acc, sden, bias, g, beta, wr_i, wz_i, wn_i,
               br_i, bz_i, bn_i, br_h, bz_h, bn_h, wh, bh, out):
    x = _ln(_combine(acc[...], sden[...], bias[...]), g[...], beta[...])
    ir = jnp.dot(x, wr_i[...], preferred_element_type=F32) + br_i[...]
    iz = jnp.dot(x, wz_i[...], preferred_element_type=F32) + bz_i[...]
    inn = jnp.dot(x, wn_i[...], preferred_element_type=F32) + bn_i[...]
    r = jax.nn.sigmoid(ir + br_h[...])
    z = jax.nn.sigmoid(iz + bz_h[...])
    nn_ = jnp.tanh(inn + r * bn_h[...])
    h = (1.0 - z) * nn_
    y = jnp.dot(h, wh[...], preferred_element_type=F32) + bh[...]
    is_relu = lax.broadcasted_iota(I32, y.shape, 1) == 2
    out[...] = jnp.where(is_relu, jnp.maximum(y, 0.0), jax.nn.sigmoid(y))


def _tc_c(acc, sden, bias, g, beta, wr_i, wz_i, wn_i,
          br_i, bz_i, bn_i, br_h, bz_h, bn_h, wh, bh):
    full = lambda shp: pl.BlockSpec(shp, lambda i: tuple(0 for _ in shp))
    return pl.pallas_call(
        _tc_c_body,
        grid=(GRID,),
        in_specs=[pl.BlockSpec((2, R, CH), lambda i: (0, i, 0)),
                  pl.BlockSpec((2, R), lambda i: (0, i)),
                  full((1, HID)), full((1, HID)), full((1, HID)),
                  full((HID, HID)), full((HID, HID)), full((HID, HID)),
                  full((1, HID)), full((1, HID)), full((1, HID)),
                  full((1, HID)), full((1, HID)), full((1, HID)),
                  full((HID, 4)), full((1, 4))],
        out_specs=pl.BlockSpec((R, 4), lambda i: (i, 0)),
        out_shape=jax.ShapeDtypeStruct((NPAD, 4), F32),
    )(acc, sden, bias, g, beta, wr_i, wz_i, wn_i,
      br_i, bz_i, bn_i, br_h, bz_h, bn_h, wh, bh)


def kernel(node_features, edge_index, W_proj, b_proj, Wl1, bl1, Wr1, br1,
           att1, bias1, Wl2, bl2, Wr2, br2, att2, bias2, g1, beta1, g2,
           beta2, W_ih, b_ih, W_hh, b_hh, Wu, bu, Wf, bf, Wo, bo, Wc, bc):
    nf_p = jnp.zeros((NPAD, 16), F32).at[:N, :NODE_DIM].set(node_features)
    wp = jnp.zeros((16, HID), F32).at[:NODE_DIM].set(W_proj.T)

    loops = jnp.arange(N, dtype=I32)
    padi = jnp.full((EPAD - ETOT,), NPAD - 1, I32)
    srcp = jnp.concatenate([edge_index[0], loops, padi])
    dstp = jnp.concatenate([edge_index[1], loops, padi])

    row = lambda v: v.reshape(1, -1)

    xl1, xr1 = _tc_a(nf_p, wp, row(b_proj), Wl1.T, row(bl1), Wr1.T, row(br1))
    acc1, s1 = _sc_edge(xl1.reshape(2 * NPAD, CH), xr1.reshape(2 * NPAD, CH),
                        srcp, dstp, att1)
    xl2, xr2 = _tc_b(acc1, s1, row(bias1), row(g1), row(beta1),
                     Wl2.T, row(bl2), Wr2.T, row(br2))
    acc2, s2 = _sc_edge(xl2.reshape(2 * NPAD, CH), xr2.reshape(2 * NPAD, CH),
                        srcp, dstp, att2)

    wiT = W_ih.T
    bi = b_ih
    bh_ = b_hh
    wh = jnp.concatenate([Wu, Wf, Wo, Wc], axis=0).T
    bhead = jnp.concatenate([bu, bf, bo, bc]).reshape(1, 4)
    out4 = _tc_c(acc2, s2, row(bias2), row(g2), row(beta2),
                 wiT[:, :HID], wiT[:, HID:2 * HID], wiT[:, 2 * HID:],
                 row(bi[:HID]), row(bi[HID:2 * HID]), row(bi[2 * HID:]),
                 row(bh_[:HID]), row(bh_[HID:2 * HID]), row(bh_[2 * HID:]),
                 wh, bhead)

    util = out4[:N, 0:1]
    feas = out4[:N, 1:2]
    ot = out4[:N, 2:3]
    conf = out4[:N, 3:4]
    return (util, feas, ot, conf)
